# Initial kernel scaffold; baseline (speedup 1.0000x reference)
import jax
import jax.numpy as jnp
from jax.experimental import pallas as pl


def kernel(drug_x, drug_edge, batch, target_ddi_index, W_init, b_init, W_l, b_l, W_r, conv_w, conv_b, W1, b1, W2, b2):
    raise NotImplementedError("write your pallas kernel here")


# speedup vs baseline: 1.8406x; 1.8406x over previous
import functools

import jax
import jax.numpy as jnp
from jax import lax
from jax.experimental import pallas as pl
from jax.experimental.pallas import tpu as pltpu
from jax.experimental.pallas import tpu_sc as plsc

N = 50000
E = 800000
G = 2048
P = 16384
IN = 75
H = 64

NSC = 2
NTEC = 16
ROWS_TEC = 1568
ROWS_SC = NTEC * ROWS_TEC
NPAD = NSC * ROWS_SC
ACC_ROWS = ROWS_SC + 8
CH = 128
NCHUNK = 391
EPT = NCHUNK * CH
EPAD = EPT * NTEC

BLK = 512
HI = lax.Precision.HIGHEST


def _h_body(x_ref, w_ref, b_ref, o_ref):
    o_ref[...] = jnp.dot(x_ref[...], w_ref[...], precision=HI) + b_ref[...]


def _init_transform(xp, W, b):
    return pl.pallas_call(
        _h_body,
        grid=(NPAD // BLK,),
        in_specs=[pl.BlockSpec((BLK, IN), lambda i: (i, 0)),
                  pl.BlockSpec((IN, H), lambda i: (0, 0)),
                  pl.BlockSpec((1, H), lambda i: (0, 0))],
        out_specs=pl.BlockSpec((BLK, H), lambda i: (i, 0)),
        out_shape=jax.ShapeDtypeStruct((NPAD, H), jnp.float32),
    )(xp, W, b.reshape(1, H))


def _seg_sum_sc(h_pad, src_pad, dst_pad, zrows):
    mesh = plsc.VectorSubcoreMesh(core_axis_name="c", subcore_axis_name="s")

    @functools.partial(
        pl.kernel,
        mesh=mesh,
        compiler_params=pltpu.CompilerParams(use_tc_tiling_on_sc=False),
        out_type=jax.ShapeDtypeStruct((NPAD, H), jnp.float32),
        scratch_types=[
            pltpu.VMEM((CH,), jnp.int32),
            pltpu.VMEM((CH,), jnp.int32),
            pltpu.VMEM((CH,), jnp.int32),
            pltpu.VMEM((CH, H), jnp.float32),
            pltpu.VMEM_SHARED((ACC_ROWS, H), jnp.float32),
            pltpu.SemaphoreType.DMA,
        ],
    )
    def k(h_hbm, src_hbm, dst_hbm, z_hbm, out_hbm,
          src_v, dst_v, dstl_v, rows_v, acc, sem):
        c = lax.axis_index("c")
        s = lax.axis_index("s")
        base = c * ROWS_SC

        pltpu.sync_copy(z_hbm, acc.at[pl.ds(s * ROWS_TEC, ROWS_TEC)])

        @pl.when(s == 0)
        def _():
            pltpu.sync_copy(z_hbm.at[pl.ds(0, 8)], acc.at[pl.ds(ROWS_SC, 8)])

        plsc.subcore_barrier()

        def body(i, carry):
            off = s * EPT + i * CH
            pltpu.sync_copy(src_hbm.at[pl.ds(off, CH)], src_v)
            pltpu.sync_copy(dst_hbm.at[pl.ds(off, CH)], dst_v)
            for j in range(CH // 16):
                d = dst_v[pl.ds(j * 16, 16)]
                inr = (d >= base) & (d < base + ROWS_SC)
                dstl_v[pl.ds(j * 16, 16)] = jnp.where(inr, d - base, ROWS_SC)
            pltpu.async_copy(h_hbm.at[src_v], rows_v, sem).wait()
            pltpu.sync_copy(rows_v, acc.at[dstl_v], add=True)
            return carry

        lax.fori_loop(0, NCHUNK, body, 0)
        plsc.subcore_barrier()
        pltpu.sync_copy(acc.at[pl.ds(s * ROWS_TEC, ROWS_TEC)],
                        out_hbm.at[pl.ds(base + s * ROWS_TEC, ROWS_TEC)])

    return k(h_pad, src_pad, dst_pad, zrows)


def _sage_pool_body(agg_ref, h_ref, wl_ref, bl_ref, wr_ref, b_ref,
                    g_ref, gs, cnt):
    i = pl.program_id(0)
    nb = pl.num_programs(0)

    @pl.when(i == 0)
    def _():
        gs[...] = jnp.zeros_like(gs)
        cnt[...] = jnp.zeros_like(cnt)

    h2 = (jnp.dot(agg_ref[...], wl_ref[...], precision=HI) + bl_ref[...]
          + jnp.dot(h_ref[...], wr_ref[...], precision=HI))
    bvec = b_ref[0, 0, :]
    onehot = (bvec[:, None] ==
              lax.broadcasted_iota(jnp.int32, (BLK, G), 1)).astype(jnp.float32)
    gs[...] += lax.dot_general(onehot, h2, (((0,), (0,)), ((), ())),
                               precision=HI)
    cnt[...] += lax.dot_general(onehot, jnp.ones((BLK, H), jnp.float32),
                                (((0,), (0,)), ((), ())), precision=HI)

    @pl.when(i == nb - 1)
    def _():
        g_ref[...] = gs[...] / jnp.maximum(cnt[...], 1.0)


def _sage_pool(aggp, hp, W_l, b_l, W_r, batch_r):
    return pl.pallas_call(
        _sage_pool_body,
        grid=(NPAD // BLK,),
        in_specs=[pl.BlockSpec((BLK, H), lambda i: (i, 0)),
                  pl.BlockSpec((BLK, H), lambda i: (i, 0)),
                  pl.BlockSpec((H, H), lambda i: (0, 0)),
                  pl.BlockSpec((1, H), lambda i: (0, 0)),
                  pl.BlockSpec((H, H), lambda i: (0, 0)),
                  pl.BlockSpec((1, 1, BLK), lambda i: (i, 0, 0))],
        out_specs=pl.BlockSpec((G, H), lambda i: (0, 0)),
        out_shape=jax.ShapeDtypeStruct((G, H), jnp.float32),
        scratch_shapes=[pltpu.VMEM((G, H), jnp.float32),
                        pltpu.VMEM((G, H), jnp.float32)],
    )(aggp, hp, W_l, b_l.reshape(1, H), W_r, batch_r)


def _decoder_body(g_ref, i0_ref, i1_ref, ce1, ce2, co1, co2, be_ref,
                  w1, b1r, w2, b2r, o_ref):
    gv = g_ref[...]
    i0 = i0_ref[0, 0, :]
    i1 = i1_ref[0, 0, :]
    iota = lax.broadcasted_iota(jnp.int32, (BLK, G), 1)
    oh0 = (i0[:, None] == iota).astype(jnp.float32)
    oh1 = (i1[:, None] == iota).astype(jnp.float32)
    x1 = jnp.dot(oh0, gv, precision=HI)
    x2 = jnp.dot(oh1, gv, precision=HI)
    ye = (jnp.dot(x1, ce1[...], precision=HI)
          + jnp.dot(x2, ce2[...], precision=HI))
    yo = (jnp.dot(x1, co1[...], precision=HI)
          + jnp.dot(x2, co2[...], precision=HI))
    z = jnp.maximum(ye, yo) + be_ref[...]
    a = jnp.maximum(jnp.dot(z, w1[...], precision=HI) + b1r[...], 0.0)
    o = jnp.dot(a, w2[...], precision=HI) + b2r[...]
    o_ref[...] = jax.nn.sigmoid(o)


def _decoder(g, i0_r, i1_r, Ce1, Ce2, Co1, Co2, be, W1, b1, W2, b2):
    C4 = 4 * H
    return pl.pallas_call(
        _decoder_body,
        grid=(P // BLK,),
        in_specs=[pl.BlockSpec((G, H), lambda i: (0, 0)),
                  pl.BlockSpec((1, 1, BLK), lambda i: (i, 0, 0)),
                  pl.BlockSpec((1, 1, BLK), lambda i: (i, 0, 0)),
                  pl.BlockSpec((H, C4), lambda i: (0, 0)),
                  pl.BlockSpec((H, C4), lambda i: (0, 0)),
                  pl.BlockSpec((H, C4), lambda i: (0, 0)),
                  pl.BlockSpec((H, C4), lambda i: (0, 0)),
                  pl.BlockSpec((1, C4), lambda i: (0, 0)),
                  pl.BlockSpec((C4, H), lambda i: (0, 0)),
                  pl.BlockSpec((1, H), lambda i: (0, 0)),
                  pl.BlockSpec((H, 1), lambda i: (0, 0)),
                  pl.BlockSpec((1, 1), lambda i: (0, 0))],
        out_specs=pl.BlockSpec((BLK, 1), lambda i: (i, 0)),
        out_shape=jax.ShapeDtypeStruct((P, 1), jnp.float32),
    )(g, i0_r, i1_r, Ce1, Ce2, Co1, Co2, be, W1,
      b1.reshape(1, H), W2, b2.reshape(1, 1))


def _conv_mats(conv_w, conv_b):
    w3 = conv_w[:, 0, :]
    k_idx = jnp.arange(H)
    Ce = jnp.zeros((2 * H, 4 * H), jnp.float32)
    Co = jnp.zeros((2 * H, 4 * H), jnp.float32)
    for o in range(4):
        for t in range(3):
            ie = 2 * k_idx + t - 1
            ve = (ie >= 0) & (ie < 2 * H)
            Ce = Ce.at[jnp.clip(ie, 0, 2 * H - 1), o * H + k_idx].add(
                jnp.where(ve, w3[o, t], 0.0))
            io_ = 2 * k_idx + t
            vo = io_ < 2 * H
            Co = Co.at[jnp.clip(io_, 0, 2 * H - 1), o * H + k_idx].add(
                jnp.where(vo, w3[o, t], 0.0))
    be = jnp.repeat(conv_b, H).reshape(1, 4 * H)
    return Ce, Co, be


def kernel(drug_x, drug_edge, batch, target_ddi_index,
           W_init, b_init, W_l, b_l, W_r, conv_w, conv_b, W1, b1, W2, b2):
    xp = jnp.pad(drug_x, ((0, NPAD - N), (0, 0)))
    src = jnp.pad(drug_edge[0], (0, EPAD - E))
    dst = jnp.pad(drug_edge[1], (0, EPAD - E), constant_values=NPAD)
    zrows = jnp.zeros((ROWS_TEC, H), jnp.float32)
    batch_r = jnp.pad(batch, (0, NPAD - N),
                      constant_values=G).reshape(NPAD // BLK, 1, BLK)
    i0_r = target_ddi_index[0].reshape(P // BLK, 1, BLK)
    i1_r = target_ddi_index[1].reshape(P // BLK, 1, BLK)
    Ce, Co, be = _conv_mats(conv_w, conv_b)

    hp = _init_transform(xp, W_init, b_init)
    aggp = _seg_sum_sc(hp, src, dst, zrows)
    g = _sage_pool(aggp, hp, W_l, b_l, W_r, batch_r)
    ddi = _decoder(g, i0_r, i1_r, Ce[:H], Ce[H:], Co[:H], Co[H:],
                   be, W1, b1, W2, b2)
    return (g, ddi.reshape(-1))


# speedup vs baseline: 2.0043x; 1.0889x over previous
import functools

import jax
import jax.numpy as jnp
from jax import lax
from jax.experimental import pallas as pl
from jax.experimental.pallas import tpu as pltpu
from jax.experimental.pallas import tpu_sc as plsc

N = 50000
E = 800000
G = 2048
P = 16384
IN = 75
H = 64

NSC = 2
NTEC = 16
ROWS_TEC = 1568
ROWS_SC = NTEC * ROWS_TEC
NPAD = NSC * ROWS_SC
ACC_ROWS = ROWS_SC + 8
CH = 128
NCHUNK = 391
EPT = NCHUNK * CH
EPAD = EPT * NTEC

BLK = 512
HI = lax.Precision.HIGHEST


def _h_body(x_ref, w_ref, b_ref, o_ref):
    o_ref[...] = jnp.dot(x_ref[...], w_ref[...], precision=HI) + b_ref[...]


def _init_transform(xp, W, b):
    return pl.pallas_call(
        _h_body,
        grid=(NPAD // BLK,),
        in_specs=[pl.BlockSpec((BLK, IN), lambda i: (i, 0)),
                  pl.BlockSpec((IN, H), lambda i: (0, 0)),
                  pl.BlockSpec((1, H), lambda i: (0, 0))],
        out_specs=pl.BlockSpec((BLK, H), lambda i: (i, 0)),
        out_shape=jax.ShapeDtypeStruct((NPAD, H), jnp.float32),
    )(xp, W, b.reshape(1, H))


def _seg_sum_sc(h_pad, src_pad, dst_pad, zrows):
    mesh = plsc.VectorSubcoreMesh(core_axis_name="c", subcore_axis_name="s")

    @functools.partial(
        pl.kernel,
        mesh=mesh,
        compiler_params=pltpu.CompilerParams(use_tc_tiling_on_sc=False,
                                             needs_layout_passes=False),
        out_type=jax.ShapeDtypeStruct((NPAD, H), jnp.float32),
        scratch_types=[
            pltpu.VMEM((CH,), jnp.int32),
            pltpu.VMEM((CH,), jnp.int32),
            pltpu.VMEM((2 * CH + 32,), jnp.int32),
            pltpu.VMEM((2 * CH + 32,), jnp.int32),
            pltpu.VMEM((CH, H), jnp.float32),
            pltpu.VMEM_SHARED((ACC_ROWS, H), jnp.float32),
            pltpu.SemaphoreType.DMA,
        ],
    )
    def k(h_hbm, src_hbm, dst_hbm, z_hbm, out_hbm,
          src_v, dst_v, psrc, pdstl, rows_v, acc, sem):
        c = lax.axis_index("c")
        s = lax.axis_index("s")
        base = c * ROWS_SC

        pltpu.sync_copy(z_hbm, acc.at[pl.ds(s * ROWS_TEC, ROWS_TEC)])

        @pl.when(s == 0)
        def _():
            pltpu.sync_copy(z_hbm.at[pl.ds(0, 8)], acc.at[pl.ds(ROWS_SC, 8)])

        plsc.subcore_barrier()

        def fire():
            pltpu.async_copy(h_hbm.at[psrc.at[pl.ds(0, CH)]], rows_v,
                             sem).wait()
            pltpu.sync_copy(rows_v, acc.at[pdstl.at[pl.ds(0, CH)]], add=True)
            for j in range(CH // 16):
                psrc[pl.ds(j * 16, 16)] = psrc[pl.ds(CH + j * 16, 16)]
                pdstl[pl.ds(j * 16, 16)] = pdstl[pl.ds(CH + j * 16, 16)]

        def body(i, n):
            off = s * EPT + i * CH
            pltpu.sync_copy(src_hbm.at[pl.ds(off, CH)], src_v)
            pltpu.sync_copy(dst_hbm.at[pl.ds(off, CH)], dst_v)
            lane = lax.iota(jnp.int32, 16)
            for j in range(CH // 16):
                d = dst_v[pl.ds(j * 16, 16)]
                m = (d >= base) & (d < base + ROWS_SC)
                mi = m.astype(jnp.int32)
                cum = plsc.cumsum(mi)
                pos = jnp.where(m, n + cum - 1, 2 * CH + lane)
                plsc.store_scatter(psrc, [pos], src_v[pl.ds(j * 16, 16)])
                plsc.store_scatter(pdstl, [pos], d - base)
                n = n + jnp.sum(mi)

            def do_fire(nn):
                fire()
                return nn - CH

            return lax.cond(n >= CH, do_fire, lambda nn: nn, n)

        n = lax.fori_loop(0, NCHUNK, body, 0)
        for j in range(CH // 16):
            psrc[pl.ds(n + j * 16, 16)] = jnp.zeros((16,), jnp.int32)
            pdstl[pl.ds(n + j * 16, 16)] = jnp.full((16,), ROWS_SC, jnp.int32)
        fire()
        plsc.subcore_barrier()
        pltpu.sync_copy(acc.at[pl.ds(s * ROWS_TEC, ROWS_TEC)],
                        out_hbm.at[pl.ds(base + s * ROWS_TEC, ROWS_TEC)])

    return k(h_pad, src_pad, dst_pad, zrows)


def _sage_pool_body(agg_ref, h_ref, wl_ref, bl_ref, wr_ref, b_ref,
                    g_ref, gs, cnt):
    i = pl.program_id(0)
    nb = pl.num_programs(0)

    @pl.when(i == 0)
    def _():
        gs[...] = jnp.zeros_like(gs)
        cnt[...] = jnp.zeros_like(cnt)

    h2 = (jnp.dot(agg_ref[...], wl_ref[...], precision=HI) + bl_ref[...]
          + jnp.dot(h_ref[...], wr_ref[...], precision=HI))
    bvec = b_ref[0, 0, :]
    onehot = (bvec[:, None] ==
              lax.broadcasted_iota(jnp.int32, (BLK, G), 1)).astype(jnp.float32)
    gs[...] += lax.dot_general(onehot, h2, (((0,), (0,)), ((), ())),
                               precision=HI)
    cnt[...] += lax.dot_general(onehot, jnp.ones((BLK, H), jnp.float32),
                                (((0,), (0,)), ((), ())), precision=HI)

    @pl.when(i == nb - 1)
    def _():
        g_ref[...] = gs[...] / jnp.maximum(cnt[...], 1.0)


def _sage_pool(aggp, hp, W_l, b_l, W_r, batch_r):
    return pl.pallas_call(
        _sage_pool_body,
        grid=(NPAD // BLK,),
        in_specs=[pl.BlockSpec((BLK, H), lambda i: (i, 0)),
                  pl.BlockSpec((BLK, H), lambda i: (i, 0)),
                  pl.BlockSpec((H, H), lambda i: (0, 0)),
                  pl.BlockSpec((1, H), lambda i: (0, 0)),
                  pl.BlockSpec((H, H), lambda i: (0, 0)),
                  pl.BlockSpec((1, 1, BLK), lambda i: (i, 0, 0))],
        out_specs=pl.BlockSpec((G, H), lambda i: (0, 0)),
        out_shape=jax.ShapeDtypeStruct((G, H), jnp.float32),
        scratch_shapes=[pltpu.VMEM((G, H), jnp.float32),
                        pltpu.VMEM((G, H), jnp.float32)],
    )(aggp, hp, W_l, b_l.reshape(1, H), W_r, batch_r)


def _decoder_body(g_ref, i0_ref, i1_ref, ce1, ce2, co1, co2, be_ref,
                  w1, b1r, w2, b2r, o_ref):
    gv = g_ref[...]
    i0 = i0_ref[0, 0, :]
    i1 = i1_ref[0, 0, :]
    iota = lax.broadcasted_iota(jnp.int32, (BLK, G), 1)
    oh0 = (i0[:, None] == iota).astype(jnp.float32)
    oh1 = (i1[:, None] == iota).astype(jnp.float32)
    x1 = jnp.dot(oh0, gv, precision=HI)
    x2 = jnp.dot(oh1, gv, precision=HI)
    ye = (jnp.dot(x1, ce1[...], precision=HI)
          + jnp.dot(x2, ce2[...], precision=HI))
    yo = (jnp.dot(x1, co1[...], precision=HI)
          + jnp.dot(x2, co2[...], precision=HI))
    z = jnp.maximum(ye, yo) + be_ref[...]
    a = jnp.maximum(jnp.dot(z, w1[...], precision=HI) + b1r[...], 0.0)
    o = jnp.dot(a, w2[...], precision=HI) + b2r[...]
    o_ref[...] = jax.nn.sigmoid(o)


def _decoder(g, i0_r, i1_r, Ce1, Ce2, Co1, Co2, be, W1, b1, W2, b2):
    C4 = 4 * H
    return pl.pallas_call(
        _decoder_body,
        grid=(P // BLK,),
        in_specs=[pl.BlockSpec((G, H), lambda i: (0, 0)),
                  pl.BlockSpec((1, 1, BLK), lambda i: (i, 0, 0)),
                  pl.BlockSpec((1, 1, BLK), lambda i: (i, 0, 0)),
                  pl.BlockSpec((H, C4), lambda i: (0, 0)),
                  pl.BlockSpec((H, C4), lambda i: (0, 0)),
                  pl.BlockSpec((H, C4), lambda i: (0, 0)),
                  pl.BlockSpec((H, C4), lambda i: (0, 0)),
                  pl.BlockSpec((1, C4), lambda i: (0, 0)),
                  pl.BlockSpec((C4, H), lambda i: (0, 0)),
                  pl.BlockSpec((1, H), lambda i: (0, 0)),
                  pl.BlockSpec((H, 1), lambda i: (0, 0)),
                  pl.BlockSpec((1, 1), lambda i: (0, 0))],
        out_specs=pl.BlockSpec((BLK, 1), lambda i: (i, 0)),
        out_shape=jax.ShapeDtypeStruct((P, 1), jnp.float32),
    )(g, i0_r, i1_r, Ce1, Ce2, Co1, Co2, be, W1,
      b1.reshape(1, H), W2, b2.reshape(1, 1))


def _conv_mats(conv_w, conv_b):
    w3 = conv_w[:, 0, :]
    k_idx = jnp.arange(H)
    Ce = jnp.zeros((2 * H, 4 * H), jnp.float32)
    Co = jnp.zeros((2 * H, 4 * H), jnp.float32)
    for o in range(4):
        for t in range(3):
            ie = 2 * k_idx + t - 1
            ve = (ie >= 0) & (ie < 2 * H)
            Ce = Ce.at[jnp.clip(ie, 0, 2 * H - 1), o * H + k_idx].add(
                jnp.where(ve, w3[o, t], 0.0))
            io_ = 2 * k_idx + t
            vo = io_ < 2 * H
            Co = Co.at[jnp.clip(io_, 0, 2 * H - 1), o * H + k_idx].add(
                jnp.where(vo, w3[o, t], 0.0))
    be = jnp.repeat(conv_b, H).reshape(1, 4 * H)
    return Ce, Co, be


def kernel(drug_x, drug_edge, batch, target_ddi_index,
           W_init, b_init, W_l, b_l, W_r, conv_w, conv_b, W1, b1, W2, b2):
    xp = jnp.pad(drug_x, ((0, NPAD - N), (0, 0)))
    src = jnp.pad(drug_edge[0], (0, EPAD - E))
    dst = jnp.pad(drug_edge[1], (0, EPAD - E), constant_values=NPAD)
    zrows = jnp.zeros((ROWS_TEC, H), jnp.float32)
    batch_r = jnp.pad(batch, (0, NPAD - N),
                      constant_values=G).reshape(NPAD // BLK, 1, BLK)
    i0_r = target_ddi_index[0].reshape(P // BLK, 1, BLK)
    i1_r = target_ddi_index[1].reshape(P // BLK, 1, BLK)
    Ce, Co, be = _conv_mats(conv_w, conv_b)

    hp = _init_transform(xp, W_init, b_init)
    aggp = _seg_sum_sc(hp, src, dst, zrows)
    g = _sage_pool(aggp, hp, W_l, b_l, W_r, batch_r)
    ddi = _decoder(g, i0_r, i1_r, Ce[:H], Ce[H:], Co[:H], Co[H:],
                   be, W1, b1, W2, b2)
    return (g, ddi.reshape(-1))


# speedup vs baseline: 3.9254x; 1.9585x over previous
import functools

import jax
import jax.numpy as jnp
from jax import lax
from jax.experimental import pallas as pl
from jax.experimental.pallas import tpu as pltpu
from jax.experimental.pallas import tpu_sc as plsc

N = 50000
E = 800000
G = 2048
P = 16384
IN = 75
H = 64

NSC = 2
NTEC = 16
ROWS_TEC = 1568
ROWS_SC = NTEC * ROWS_TEC
NPAD = NSC * ROWS_SC
ACC_ROWS = ROWS_SC + 8
CH = 128
NCHUNK = 391
EPT = NCHUNK * CH
EPAD = EPT * NTEC

HE = H + 16
GP = G + 16

BLK = 1024
BLKD = 512
HI = lax.Precision.HIGHEST

_SC_PARAMS = pltpu.CompilerParams(use_tc_tiling_on_sc=False,
                                  needs_layout_passes=False)
_SC_MESH = dict(core_axis_name="c", subcore_axis_name="s")


def _h_body(x_ref, w_ref, b_ref, o_ref):
    o_ref[...] = jnp.dot(x_ref[...], w_ref[...], precision=HI) + b_ref[...]


def _init_transform(xp, W, b):
    return pl.pallas_call(
        _h_body,
        grid=(NPAD // BLK,),
        in_specs=[pl.BlockSpec((BLK, IN), lambda i: (i, 0)),
                  pl.BlockSpec((IN, H), lambda i: (0, 0)),
                  pl.BlockSpec((1, H), lambda i: (0, 0))],
        out_specs=pl.BlockSpec((BLK, H), lambda i: (i, 0)),
        out_shape=jax.ShapeDtypeStruct((NPAD, H), jnp.float32),
    )(xp, W, b.reshape(1, H))


def _seg_sum_sc(h_pad, src_pad, dst_pad, zrows):
    mesh = plsc.VectorSubcoreMesh(**_SC_MESH)

    @functools.partial(
        pl.kernel,
        mesh=mesh,
        compiler_params=_SC_PARAMS,
        out_type=jax.ShapeDtypeStruct((NPAD, H), jnp.float32),
        scratch_types=[
            pltpu.VMEM((CH,), jnp.int32),
            pltpu.VMEM((CH,), jnp.int32),
            pltpu.VMEM((2 * CH + 32,), jnp.int32),
            pltpu.VMEM((2 * CH + 32,), jnp.int32),
            pltpu.VMEM((CH, H), jnp.float32),
            pltpu.VMEM_SHARED((ACC_ROWS, H), jnp.float32),
            pltpu.SemaphoreType.DMA,
        ],
    )
    def k(h_hbm, src_hbm, dst_hbm, z_hbm, out_hbm,
          src_v, dst_v, psrc, pdstl, rows_v, acc, sem):
        c = lax.axis_index("c")
        s = lax.axis_index("s")
        base = c * ROWS_SC

        pltpu.sync_copy(z_hbm, acc.at[pl.ds(s * ROWS_TEC, ROWS_TEC)])

        @pl.when(s == 0)
        def _():
            pltpu.sync_copy(z_hbm.at[pl.ds(0, 8)], acc.at[pl.ds(ROWS_SC, 8)])

        plsc.subcore_barrier()

        def fire():
            pltpu.async_copy(h_hbm.at[psrc.at[pl.ds(0, CH)]], rows_v,
                             sem).wait()
            pltpu.sync_copy(rows_v, acc.at[pdstl.at[pl.ds(0, CH)]], add=True)
            for j in range(CH // 16):
                psrc[pl.ds(j * 16, 16)] = psrc[pl.ds(CH + j * 16, 16)]
                pdstl[pl.ds(j * 16, 16)] = pdstl[pl.ds(CH + j * 16, 16)]

        def body(i, n):
            off = s * EPT + i * CH
            pltpu.sync_copy(src_hbm.at[pl.ds(off, CH)], src_v)
            pltpu.sync_copy(dst_hbm.at[pl.ds(off, CH)], dst_v)
            lane = lax.iota(jnp.int32, 16)
            for j in range(CH // 16):
                d = dst_v[pl.ds(j * 16, 16)]
                m = (d >= base) & (d < base + ROWS_SC)
                mi = m.astype(jnp.int32)
                cum = plsc.cumsum(mi)
                pos = jnp.where(m, n + cum - 1, 2 * CH + lane)
                plsc.store_scatter(psrc, [pos], src_v[pl.ds(j * 16, 16)])
                plsc.store_scatter(pdstl, [pos], d - base)
                n = n + jnp.sum(mi)

            def do_fire(nn):
                fire()
                return nn - CH

            return lax.cond(n >= CH, do_fire, lambda nn: nn, n)

        n = lax.fori_loop(0, NCHUNK, body, 0)
        for j in range(CH // 16):
            psrc[pl.ds(n + j * 16, 16)] = jnp.zeros((16,), jnp.int32)
            pdstl[pl.ds(n + j * 16, 16)] = jnp.full((16,), ROWS_SC, jnp.int32)
        fire()
        plsc.subcore_barrier()
        pltpu.sync_copy(acc.at[pl.ds(s * ROWS_TEC, ROWS_TEC)],
                        out_hbm.at[pl.ds(base + s * ROWS_TEC, ROWS_TEC)])

    return k(h_pad, src_pad, dst_pad, zrows)


def _h2_body(agg_ref, h_ref, wl_ref, bl_ref, wr_ref, o_ref):
    h2 = (jnp.dot(agg_ref[...], wl_ref[...], precision=HI) + bl_ref[...]
          + jnp.dot(h_ref[...], wr_ref[...], precision=HI))
    ext = (lax.broadcasted_iota(jnp.int32, (BLK, HE - H), 1) == 0
           ).astype(jnp.float32)
    o_ref[...] = jnp.concatenate([h2, ext], axis=1)


def _h2_ext(aggp, hp, W_l, b_l, W_r):
    return pl.pallas_call(
        _h2_body,
        grid=(NPAD // BLK,),
        in_specs=[pl.BlockSpec((BLK, H), lambda i: (i, 0)),
                  pl.BlockSpec((BLK, H), lambda i: (i, 0)),
                  pl.BlockSpec((H, H), lambda i: (0, 0)),
                  pl.BlockSpec((1, H), lambda i: (0, 0)),
                  pl.BlockSpec((H, H), lambda i: (0, 0))],
        out_specs=pl.BlockSpec((BLK, HE), lambda i: (i, 0)),
        out_shape=jax.ShapeDtypeStruct((NPAD, HE), jnp.float32),
    )(aggp, hp, W_l, b_l.reshape(1, H), W_r)


def _pool_sc(h2e, batch_pad, zpool):
    mesh = plsc.VectorSubcoreMesh(**_SC_MESH)
    n_full = ROWS_TEC // CH
    tail = ROWS_TEC - n_full * CH

    @functools.partial(
        pl.kernel,
        mesh=mesh,
        compiler_params=_SC_PARAMS,
        out_type=jax.ShapeDtypeStruct((NSC, G, HE), jnp.float32),
        scratch_types=[
            pltpu.VMEM((CH,), jnp.int32),
            pltpu.VMEM((CH, HE), jnp.float32),
            pltpu.VMEM((tail,), jnp.int32),
            pltpu.VMEM((tail, HE), jnp.float32),
            pltpu.VMEM_SHARED((GP, HE), jnp.float32),
        ],
    )
    def k(h2_hbm, b_hbm, z_hbm, out_hbm, bidx, rows, bidx2, rows2, gs):
        c = lax.axis_index("c")
        s = lax.axis_index("s")

        pltpu.sync_copy(z_hbm, gs.at[pl.ds(s * (GP // NTEC), GP // NTEC)])
        plsc.subcore_barrier()

        def body(i, carry):
            off = (c * NTEC + s) * ROWS_TEC + i * CH
            pltpu.sync_copy(h2_hbm.at[pl.ds(off, CH)], rows)
            pltpu.sync_copy(b_hbm.at[pl.ds(off, CH)], bidx)
            pltpu.sync_copy(rows, gs.at[bidx], add=True)
            return carry

        lax.fori_loop(0, n_full, body, 0)
        off = (c * NTEC + s) * ROWS_TEC + n_full * CH
        pltpu.sync_copy(h2_hbm.at[pl.ds(off, tail)], rows2)
        pltpu.sync_copy(b_hbm.at[pl.ds(off, tail)], bidx2)
        pltpu.sync_copy(rows2, gs.at[bidx2], add=True)
        plsc.subcore_barrier()
        pltpu.sync_copy(gs.at[pl.ds(s * (G // NTEC), G // NTEC)],
                        out_hbm.at[c, pl.ds(s * (G // NTEC), G // NTEC)])

    return k(h2e, batch_pad, zpool)


def _g_body(a_ref, b_ref, g_ref):
    p = a_ref[...] + b_ref[...]
    g_ref[...] = p[:, :H] / jnp.maximum(p[:, H:H + 1], 1.0)


def _g_combine(p0, p1):
    return pl.pallas_call(
        _g_body,
        in_specs=[pl.BlockSpec((G, HE), lambda: (0, 0)),
                  pl.BlockSpec((G, HE), lambda: (0, 0))],
        out_specs=pl.BlockSpec((G, H), lambda: (0, 0)),
        out_shape=jax.ShapeDtypeStruct((G, H), jnp.float32),
    )(p0, p1)


def _pair_gather_sc(g, i0, i1):
    mesh = plsc.VectorSubcoreMesh(**_SC_MESH)
    per_w = P // (NSC * NTEC)
    n_ch = per_w // CH

    @functools.partial(
        pl.kernel,
        mesh=mesh,
        compiler_params=_SC_PARAMS,
        out_type=(jax.ShapeDtypeStruct((P, H), jnp.float32),
                  jax.ShapeDtypeStruct((P, H), jnp.float32)),
        scratch_types=[
            pltpu.VMEM((CH,), jnp.int32),
            pltpu.VMEM((CH, H), jnp.float32),
            pltpu.SemaphoreType.DMA,
        ],
    )
    def k(g_hbm, i0_hbm, i1_hbm, x1_hbm, x2_hbm, idx_v, rows, sem):
        c = lax.axis_index("c")
        s = lax.axis_index("s")
        w = c * NTEC + s

        def body(i, carry):
            off = w * per_w + i * CH
            pltpu.sync_copy(i0_hbm.at[pl.ds(off, CH)], idx_v)
            pltpu.async_copy(g_hbm.at[idx_v], rows, sem).wait()
            pltpu.sync_copy(rows, x1_hbm.at[pl.ds(off, CH)])
            pltpu.sync_copy(i1_hbm.at[pl.ds(off, CH)], idx_v)
            pltpu.async_copy(g_hbm.at[idx_v], rows, sem).wait()
            pltpu.sync_copy(rows, x2_hbm.at[pl.ds(off, CH)])
            return carry

        lax.fori_loop(0, n_ch, body, 0)

    return k(g, i0, i1)


def _decoder_body(x1_ref, x2_ref, ce1, ce2, co1, co2, be_ref,
                  w1, b1r, w2, b2r, o_ref):
    x1 = x1_ref[...]
    x2 = x2_ref[...]
    ye = (jnp.dot(x1, ce1[...], precision=HI)
          + jnp.dot(x2, ce2[...], precision=HI))
    yo = (jnp.dot(x1, co1[...], precision=HI)
          + jnp.dot(x2, co2[...], precision=HI))
    z = jnp.maximum(ye, yo) + be_ref[...]
    a = jnp.maximum(jnp.dot(z, w1[...], precision=HI) + b1r[...], 0.0)
    o = jnp.dot(a, w2[...], precision=HI) + b2r[...]
    o_ref[...] = jax.nn.sigmoid(o)


def _decoder(x1, x2, Ce1, Ce2, Co1, Co2, be, W1, b1, W2, b2):
    C4 = 4 * H
    return pl.pallas_call(
        _decoder_body,
        grid=(P // BLKD,),
        in_specs=[pl.BlockSpec((BLKD, H), lambda i: (i, 0)),
                  pl.BlockSpec((BLKD, H), lambda i: (i, 0)),
                  pl.BlockSpec((H, C4), lambda i: (0, 0)),
                  pl.BlockSpec((H, C4), lambda i: (0, 0)),
                  pl.BlockSpec((H, C4), lambda i: (0, 0)),
                  pl.BlockSpec((H, C4), lambda i: (0, 0)),
                  pl.BlockSpec((1, C4), lambda i: (0, 0)),
                  pl.BlockSpec((C4, H), lambda i: (0, 0)),
                  pl.BlockSpec((1, H), lambda i: (0, 0)),
                  pl.BlockSpec((H, 1), lambda i: (0, 0)),
                  pl.BlockSpec((1, 1), lambda i: (0, 0))],
        out_specs=pl.BlockSpec((BLKD, 1), lambda i: (i, 0)),
        out_shape=jax.ShapeDtypeStruct((P, 1), jnp.float32),
    )(x1, x2, Ce1, Ce2, Co1, Co2, be, W1,
      b1.reshape(1, H), W2, b2.reshape(1, 1))


def _conv_mats(conv_w, conv_b):
    w3 = conv_w[:, 0, :]
    i_g = jnp.arange(2 * H)[:, None]
    k_g = (jnp.arange(4 * H) % H)[None, :]
    Ce = jnp.zeros((2 * H, 4 * H), jnp.float32)
    Co = jnp.zeros((2 * H, 4 * H), jnp.float32)
    for t in range(3):
        wrow = jnp.repeat(w3[:, t], H)[None, :]
        Ce = Ce + jnp.where(i_g == 2 * k_g + t - 1, wrow, 0.0)
        Co = Co + jnp.where(i_g == 2 * k_g + t, wrow, 0.0)
    be = jnp.repeat(conv_b, H).reshape(1, 4 * H)
    return Ce, Co, be


def kernel(drug_x, drug_edge, batch, target_ddi_index,
           W_init, b_init, W_l, b_l, W_r, conv_w, conv_b, W1, b1, W2, b2):
    xp = jnp.pad(drug_x, ((0, NPAD - N), (0, 0)))
    src = jnp.pad(drug_edge[0], (0, EPAD - E))
    dst = jnp.pad(drug_edge[1], (0, EPAD - E), constant_values=NPAD)
    zrows = jnp.zeros((ROWS_TEC, H), jnp.float32)
    zpool = jnp.zeros((GP // NTEC, HE), jnp.float32)
    batch_pad = jnp.pad(batch, (0, NPAD - N), constant_values=G)
    Ce, Co, be = _conv_mats(conv_w, conv_b)

    hp = _init_transform(xp, W_init, b_init)
    aggp = _seg_sum_sc(hp, src, dst, zrows)
    h2e = _h2_ext(aggp, hp, W_l, b_l, W_r)
    pgs = _pool_sc(h2e, batch_pad, zpool)
    g = _g_combine(pgs[0], pgs[1])
    x1, x2 = _pair_gather_sc(g, target_ddi_index[0], target_ddi_index[1])
    ddi = _decoder(x1, x2, Ce[:H], Ce[H:], Co[:H], Co[H:],
                   be, W1, b1, W2, b2)
    return (g, ddi.reshape(-1))
